# baseline (speedup 1.0000x reference)
import functools

import jax
import jax.numpy as jnp
from jax import lax
from jax.experimental import pallas as pl
from jax.experimental.pallas import tpu as pltpu
from jax.experimental.pallas import tpu_sc as plsc

_B = 16384
_D = 64
_MARGIN = 1.0
_NC = 2
_NS = 16
_NW = _NC * _NS
_BPW = _B // _NW
_GROUPS = _BPW // 16


def _rsqrt(v):
    i = plsc.bitcast(v, jnp.int32)
    magic = jnp.full((16,), 0x5F3759DF, jnp.int32)
    y = plsc.bitcast(magic - lax.shift_right_logical(i, 1), jnp.float32)
    half = jnp.float32(0.5)
    three_half = jnp.float32(1.5)
    for _ in range(3):
        y = y * (three_half - half * v * y * y)
    return y


def _trans_e_sc(left_idx, right_idx, rel_idx, entity, relation):
    mesh = plsc.VectorSubcoreMesh(core_axis_name="c", subcore_axis_name="s")

    @functools.partial(
        pl.kernel,
        mesh=mesh,
        compiler_params=pltpu.CompilerParams(
            needs_layout_passes=False, use_tc_tiling_on_sc=False),
        out_type=jax.ShapeDtypeStruct((_NW * 16,), jnp.float32),
        scratch_types=[
            pltpu.VMEM((_BPW,), jnp.int32),
            pltpu.VMEM((_BPW,), jnp.int32),
            pltpu.VMEM((_BPW,), jnp.int32),
            pltpu.VMEM((_BPW, _D), jnp.float32),
            pltpu.VMEM((_BPW, _D), jnp.float32),
            pltpu.VMEM((_BPW, _D), jnp.float32),
            pltpu.VMEM((16,), jnp.float32),
            pltpu.SemaphoreType.DMA,
        ],
    )
    def _k(lidx_hbm, ridx_hbm, qidx_hbm, ent_hbm, rel_hbm, out_hbm,
           lidx_v, ridx_v, qidx_v, lrows_v, rrows_v, qrows_v, accv, sem):
        wid = lax.axis_index("s") * _NC + lax.axis_index("c")
        base = wid * _BPW
        pltpu.sync_copy(lidx_hbm.at[pl.ds(base, _BPW)], lidx_v)
        pltpu.sync_copy(ridx_hbm.at[pl.ds(base, _BPW)], ridx_v)
        pltpu.sync_copy(qidx_hbm.at[pl.ds(base, _BPW)], qidx_v)
        cp_l = pltpu.async_copy(ent_hbm.at[lidx_v], lrows_v, sem)
        cp_r = pltpu.async_copy(ent_hbm.at[ridx_v], rrows_v, sem)
        cp_q = pltpu.async_copy(rel_hbm.at[qidx_v], qrows_v, sem)
        cp_l.wait()
        cp_r.wait()
        cp_q.wait()

        lanes = lax.iota(jnp.int32, 16)
        zeros = jnp.zeros((16,), jnp.float32)
        eps2 = jnp.float32(1e-24)
        inv_cap = jnp.float32(1e12)
        margin = jnp.float32(_MARGIN)

        def group_body(g, acc):
            v_ll = zeros
            v_rr = zeros
            v_qq = zeros
            v_lr = zeros
            v_qr = zeros
            for rr in range(16):
                row = g * 16 + rr
                lc = [lrows_v[row, pl.ds(16 * c, 16)] for c in range(4)]
                rc = [rrows_v[row, pl.ds(16 * c, 16)] for c in range(4)]
                qc = [qrows_v[row, pl.ds(16 * c, 16)] for c in range(4)]
                p_ll = lc[0] * lc[0] + lc[1] * lc[1] + lc[2] * lc[2] + lc[3] * lc[3]
                p_rr = rc[0] * rc[0] + rc[1] * rc[1] + rc[2] * rc[2] + rc[3] * rc[3]
                p_qq = qc[0] * qc[0] + qc[1] * qc[1] + qc[2] * qc[2] + qc[3] * qc[3]
                p_lr = lc[0] * rc[0] + lc[1] * rc[1] + lc[2] * rc[2] + lc[3] * rc[3]
                p_qr = qc[0] * rc[0] + qc[1] * rc[1] + qc[2] * rc[2] + qc[3] * rc[3]
                here = lanes == rr
                v_ll = jnp.where(here, jnp.sum(p_ll), v_ll)
                v_rr = jnp.where(here, jnp.sum(p_rr), v_rr)
                v_qq = jnp.where(here, jnp.sum(p_qq), v_qq)
                v_lr = jnp.where(here, jnp.sum(p_lr), v_lr)
                v_qr = jnp.where(here, jnp.sum(p_qr), v_qr)
            inv_l = jnp.minimum(_rsqrt(jnp.maximum(v_ll, eps2)), inv_cap)
            inv_r = jnp.minimum(_rsqrt(jnp.maximum(v_rr, eps2)), inv_cap)
            inv_q = jnp.minimum(_rsqrt(jnp.maximum(v_qq, eps2)), inv_cap)
            simi = v_lr * inv_l * inv_r + v_qr * inv_q * inv_r
            similn = simi
            simirn = simi
            outl = similn - simi + margin
            outr = simirn - simi + margin
            costl = outl * (outl > 0).astype(jnp.float32)
            costr = outr * (outr > 0).astype(jnp.float32)
            return acc + costl + costr

        acc = lax.fori_loop(0, _GROUPS, group_body, zeros)
        accv[...] = acc
        pltpu.sync_copy(accv, out_hbm.at[pl.ds(wid * 16, 16)])

    return _k(left_idx, right_idx, rel_idx, entity, relation)


def kernel(leftEnIndices, rightEnIndices, relIndices, negLeftEnIndices,
           negRightEnIndices, entityEmbedding, relationEmbedding):
    del negLeftEnIndices, negRightEnIndices
    partials = _trans_e_sc(
        leftEnIndices.astype(jnp.int32),
        rightEnIndices.astype(jnp.int32),
        relIndices.astype(jnp.int32),
        entityEmbedding,
        relationEmbedding,
    )
    return jnp.sum(partials) / jnp.float32(_B)


# speedup vs baseline: 1.6398x; 1.6398x over previous
import functools

import jax
import jax.numpy as jnp
from jax import lax
from jax.experimental import pallas as pl
from jax.experimental.pallas import tpu as pltpu
from jax.experimental.pallas import tpu_sc as plsc

_B = 16384
_D = 64
_MARGIN = 1.0
_NC = 2
_NS = 16
_NW = _NC * _NS
_BPW = _B // _NW
_C = 32
_NCHUNK = _BPW // _C


def _rsqrt(v):
    i = plsc.bitcast(v, jnp.int32)
    magic = jnp.full((16,), 0x5F3759DF, jnp.int32)
    y = plsc.bitcast(magic - lax.shift_right_logical(i, 1), jnp.float32)
    half = jnp.float32(0.5)
    three_half = jnp.float32(1.5)
    for _ in range(3):
        y = y * (three_half - half * v * y * y)
    return y


def _trans_e_sc(left_idx, right_idx, rel_idx, entity, relation):
    mesh = plsc.VectorSubcoreMesh(core_axis_name="c", subcore_axis_name="s")

    @functools.partial(
        pl.kernel,
        mesh=mesh,
        compiler_params=pltpu.CompilerParams(needs_layout_passes=False),
        out_type=jax.ShapeDtypeStruct((_NW * 16,), jnp.float32),
        scratch_types=[
            pltpu.VMEM((_BPW,), jnp.int32),
            pltpu.VMEM((_BPW,), jnp.int32),
            pltpu.VMEM((_BPW,), jnp.int32),
            pltpu.VMEM((_C, _D), jnp.float32),
            pltpu.VMEM((_C, _D), jnp.float32),
            pltpu.VMEM((_C, _D), jnp.float32),
            pltpu.VMEM((16,), jnp.float32),
            pltpu.SemaphoreType.DMA,
        ],
    )
    def _k(lidx_hbm, ridx_hbm, qidx_hbm, ent_hbm, rel_hbm, out_hbm,
           lidx_v, ridx_v, qidx_v, lbuf, rbuf, qbuf, accv, sem):
        wid = lax.axis_index("s") * _NC + lax.axis_index("c")
        base = wid * _BPW
        pltpu.sync_copy(lidx_hbm.at[pl.ds(base, _BPW)], lidx_v)
        pltpu.sync_copy(ridx_hbm.at[pl.ds(base, _BPW)], ridx_v)
        pltpu.sync_copy(qidx_hbm.at[pl.ds(base, _BPW)], qidx_v)

        lanes = lax.iota(jnp.int32, 16)
        zeros = jnp.zeros((16,), jnp.float32)
        eps2 = jnp.float32(1e-24)
        inv_cap = jnp.float32(1e12)
        margin = jnp.float32(_MARGIN)

        def chunk_body(g, acc):
            co = g * _C
            copies = []
            for grp in range(_C // 16):
                row0 = co + grp * 16
                lrow = lidx_v[pl.ds(row0, 16)]
                rrow = ridx_v[pl.ds(row0, 16)]
                qrow = qidx_v[pl.ds(row0, 16)]
                for rr in range(16):
                    li = grp * 16 + rr
                    copies.append(
                        pltpu.async_copy(ent_hbm.at[lrow[rr]], lbuf.at[li], sem))
                    copies.append(
                        pltpu.async_copy(ent_hbm.at[rrow[rr]], rbuf.at[li], sem))
                    copies.append(
                        pltpu.async_copy(rel_hbm.at[qrow[rr]], qbuf.at[li], sem))
            for cp in copies:
                cp.wait()
            for grp in range(_C // 16):
                v_ll = zeros
                v_rr = zeros
                v_qq = zeros
                v_lr = zeros
                v_qr = zeros
                for rr in range(16):
                    li = grp * 16 + rr
                    lc = [lbuf[li, pl.ds(16 * c, 16)] for c in range(4)]
                    rc = [rbuf[li, pl.ds(16 * c, 16)] for c in range(4)]
                    qc = [qbuf[li, pl.ds(16 * c, 16)] for c in range(4)]
                    p_ll = lc[0] * lc[0] + lc[1] * lc[1] + lc[2] * lc[2] + lc[3] * lc[3]
                    p_rr = rc[0] * rc[0] + rc[1] * rc[1] + rc[2] * rc[2] + rc[3] * rc[3]
                    p_qq = qc[0] * qc[0] + qc[1] * qc[1] + qc[2] * qc[2] + qc[3] * qc[3]
                    p_lr = lc[0] * rc[0] + lc[1] * rc[1] + lc[2] * rc[2] + lc[3] * rc[3]
                    p_qr = qc[0] * rc[0] + qc[1] * rc[1] + qc[2] * rc[2] + qc[3] * rc[3]
                    here = lanes == rr
                    v_ll = jnp.where(here, jnp.sum(p_ll), v_ll)
                    v_rr = jnp.where(here, jnp.sum(p_rr), v_rr)
                    v_qq = jnp.where(here, jnp.sum(p_qq), v_qq)
                    v_lr = jnp.where(here, jnp.sum(p_lr), v_lr)
                    v_qr = jnp.where(here, jnp.sum(p_qr), v_qr)
                inv_l = jnp.minimum(_rsqrt(jnp.maximum(v_ll, eps2)), inv_cap)
                inv_r = jnp.minimum(_rsqrt(jnp.maximum(v_rr, eps2)), inv_cap)
                inv_q = jnp.minimum(_rsqrt(jnp.maximum(v_qq, eps2)), inv_cap)
                simi = v_lr * inv_l * inv_r + v_qr * inv_q * inv_r
                similn = simi
                simirn = simi
                outl = similn - simi + margin
                outr = simirn - simi + margin
                costl = outl * (outl > 0).astype(jnp.float32)
                costr = outr * (outr > 0).astype(jnp.float32)
                acc = acc + costl + costr
            return acc

        acc = lax.fori_loop(0, _NCHUNK, chunk_body, zeros)
        accv[...] = acc
        pltpu.sync_copy(accv, out_hbm.at[pl.ds(wid * 16, 16)])

    return _k(left_idx, right_idx, rel_idx, entity, relation)


def kernel(leftEnIndices, rightEnIndices, relIndices, negLeftEnIndices,
           negRightEnIndices, entityEmbedding, relationEmbedding):
    del negLeftEnIndices, negRightEnIndices
    partials = _trans_e_sc(
        leftEnIndices.astype(jnp.int32),
        rightEnIndices.astype(jnp.int32),
        relIndices.astype(jnp.int32),
        entityEmbedding,
        relationEmbedding,
    )
    return jnp.sum(partials) / jnp.float32(_B)
